# baseline (device time: 79486 ns/iter reference)
import jax
import jax.numpy as jnp
from jax import lax
from jax.experimental import pallas as pl
from jax.experimental.pallas import tpu as pltpu

N_DEV = 4


def kernel(x):
    _, m, n = x.shape
    rows = (m * 3) // 4

    def body(x_ref, out_ref, lbuf, rbuf, send_r, recv_r, send_l, recv_l):
        my = lax.axis_index("i")
        left = (my - 1) % N_DEV
        right = (my + 1) % N_DEV

        barrier_sem = pltpu.get_barrier_semaphore()
        for nbr in (left, right):
            pl.semaphore_signal(
                barrier_sem, inc=1,
                device_id=(nbr,), device_id_type=pl.DeviceIdType.MESH,
            )
        pl.semaphore_wait(barrier_sem, 2)

        snd_r = pltpu.make_async_remote_copy(
            src_ref=x_ref.at[0, pl.ds(0, rows)], dst_ref=rbuf,
            send_sem=send_r, recv_sem=recv_r,
            device_id=(right,), device_id_type=pl.DeviceIdType.MESH,
        )
        snd_l = pltpu.make_async_remote_copy(
            src_ref=x_ref.at[0, pl.ds(0, rows)], dst_ref=lbuf,
            send_sem=send_l, recv_sem=recv_l,
            device_id=(left,), device_id_type=pl.DeviceIdType.MESH,
        )
        snd_r.start()
        snd_l.start()
        snd_r.wait()
        snd_l.wait()
        out_ref[:, :] = x_ref[0, :, :]

    return pl.pallas_call(
        body,
        out_shape=jax.ShapeDtypeStruct((m, n), x.dtype),
        in_specs=[pl.BlockSpec(memory_space=pltpu.VMEM)],
        out_specs=pl.BlockSpec(memory_space=pltpu.VMEM),
        scratch_shapes=[
            pltpu.VMEM((rows, n), x.dtype),
            pltpu.VMEM((rows, n), x.dtype),
            pltpu.SemaphoreType.DMA,
            pltpu.SemaphoreType.DMA,
            pltpu.SemaphoreType.DMA,
            pltpu.SemaphoreType.DMA,
        ],
        compiler_params=pltpu.CompilerParams(collective_id=0),
    )(x)


# device time: 79361 ns/iter; 1.0016x vs baseline; 1.0016x over previous
import jax
import jax.numpy as jnp
from jax import lax
from jax.experimental import pallas as pl
from jax.experimental.pallas import tpu as pltpu

N_DEV = 4
STEPS = N_DEV - 1
ROUNDS = 2 * STEPS
K = 2


def kernel(x):
    _, m, n = x.shape
    half = m // 2
    chunk = half // N_DEV
    sub = chunk // K

    def body(x_ref, out_ref, rbuf_r, rbuf_l, send_r, recv_r, send_l, recv_l):
        my = lax.axis_index("i")
        left = (my - 1) % N_DEV
        right = (my + 1) % N_DEV

        barrier_sem = pltpu.get_barrier_semaphore()
        for nbr in (left, right):
            pl.semaphore_signal(
                barrier_sem, inc=1,
                device_id=(nbr,), device_id_type=pl.DeviceIdType.MESH,
            )
        pl.semaphore_wait(barrier_sem, 2)

        def row_r(c, k):
            return pl.ds((c % N_DEV) * chunk + k * sub, sub)

        def row_l(c, k):
            return pl.ds(half + (c % N_DEV) * chunk + k * sub, sub)

        def mk_send_r(s, k):
            src = (x_ref.at[0, row_r(my, k)] if s == 0
                   else out_ref.at[row_r(my - s, k)])
            dst = (rbuf_r.at[s, pl.ds(k * sub, sub)] if s < STEPS
                   else out_ref.at[row_r(my - s, k)])
            return pltpu.make_async_remote_copy(
                src_ref=src, dst_ref=dst,
                send_sem=send_r.at[s * K + k], recv_sem=recv_r.at[s * K + k],
                device_id=(right,), device_id_type=pl.DeviceIdType.MESH,
            )

        def mk_send_l(s, k):
            src = (x_ref.at[0, row_l(my, k)] if s == 0
                   else out_ref.at[row_l(my + s, k)])
            dst = (rbuf_l.at[s, pl.ds(k * sub, sub)] if s < STEPS
                   else out_ref.at[row_l(my + s, k)])
            return pltpu.make_async_remote_copy(
                src_ref=src, dst_ref=dst,
                send_sem=send_l.at[s * K + k], recv_sem=recv_l.at[s * K + k],
                device_id=(left,), device_id_type=pl.DeviceIdType.MESH,
            )

        def mk_recv_r(r, k):
            dst = (rbuf_r.at[r, pl.ds(k * sub, sub)] if r < STEPS
                   else out_ref.at[row_r(my - 1 - r, k)])
            return pltpu.make_async_remote_copy(
                src_ref=dst, dst_ref=dst,
                send_sem=send_r.at[r * K + k], recv_sem=recv_r.at[r * K + k],
                device_id=(right,), device_id_type=pl.DeviceIdType.MESH,
            )

        def mk_recv_l(r, k):
            dst = (rbuf_l.at[r, pl.ds(k * sub, sub)] if r < STEPS
                   else out_ref.at[row_l(my + 1 + r, k)])
            return pltpu.make_async_remote_copy(
                src_ref=dst, dst_ref=dst,
                send_sem=send_l.at[r * K + k], recv_sem=recv_l.at[r * K + k],
                device_id=(left,), device_id_type=pl.DeviceIdType.MESH,
            )

        pending = []

        for k in range(K):
            sr, sl = mk_send_r(0, k), mk_send_l(0, k)
            sr.start()
            sl.start()
            pending += [sr, sl]

        for s in range(1, ROUNDS):
            for k in range(K):
                mk_recv_r(s - 1, k).wait_recv()
                if s - 1 < STEPS:
                    out_ref[row_r(my - s, k), :] = (
                        x_ref[0, row_r(my - s, k), :]
                        + rbuf_r[s - 1, pl.ds(k * sub, sub), :])
                sr = mk_send_r(s, k)
                sr.start()
                pending.append(sr)

                mk_recv_l(s - 1, k).wait_recv()
                if s - 1 < STEPS:
                    out_ref[row_l(my + s, k), :] = (
                        x_ref[0, row_l(my + s, k), :]
                        + rbuf_l[s - 1, pl.ds(k * sub, sub), :])
                sl = mk_send_l(s, k)
                sl.start()
                pending.append(sl)

        for k in range(K):
            mk_recv_r(ROUNDS - 1, k).wait_recv()
            mk_recv_l(ROUNDS - 1, k).wait_recv()

        for d in pending:
            d.wait_send()

    return pl.pallas_call(
        body,
        out_shape=jax.ShapeDtypeStruct((m, n), x.dtype),
        in_specs=[pl.BlockSpec(memory_space=pltpu.VMEM)],
        out_specs=pl.BlockSpec(memory_space=pltpu.VMEM),
        scratch_shapes=[
            pltpu.VMEM((STEPS, chunk, n), x.dtype),
            pltpu.VMEM((STEPS, chunk, n), x.dtype),
            pltpu.SemaphoreType.DMA((ROUNDS * K,)),
            pltpu.SemaphoreType.DMA((ROUNDS * K,)),
            pltpu.SemaphoreType.DMA((ROUNDS * K,)),
            pltpu.SemaphoreType.DMA((ROUNDS * K,)),
        ],
        compiler_params=pltpu.CompilerParams(collective_id=0),
    )(x)
